# initial kernel scaffold (unmeasured)
import jax
import jax.numpy as jnp
from jax import lax
from jax.experimental import pallas as pl
from jax.experimental.pallas import tpu as pltpu

N_LAYERS = 3


def kernel(x, Win0, Wout0, Win1, Wout1, Win2, Wout2):
    b, d_half = x.shape
    h_dim = Win0.shape[1]

    def body(x_ref, win0_ref, wout0_ref, win1_ref, wout1_ref, win2_ref,
             wout2_ref, out_ref,
             hbuf, obuf, commh, commo,
             send_h, recv_h, send_o, recv_o):
        my_x = lax.axis_index("x")
        my_y = lax.axis_index("y")
        y_peer = (my_x, 1 - my_y)
        x_peer = (1 - my_x, my_y)

        barrier = pltpu.get_barrier_semaphore()
        for peer in (y_peer, x_peer):
            pl.semaphore_signal(barrier, inc=1, device_id=peer,
                                device_id_type=pl.DeviceIdType.MESH)
        pl.semaphore_wait(barrier, 2)

        wins = (win0_ref, win1_ref, win2_ref)
        wouts = (wout0_ref, wout1_ref, wout2_ref)

        x_cur = x_ref[...]
        for l in range(N_LAYERS):
            hbuf[...] = jnp.dot(x_cur, wins[l][...],
                                preferred_element_type=jnp.float32)
            rdma_h = pltpu.make_async_remote_copy(
                src_ref=hbuf,
                dst_ref=commh.at[l],
                send_sem=send_h.at[l],
                recv_sem=recv_h.at[l],
                device_id=y_peer,
                device_id_type=pl.DeviceIdType.MESH,
            )
            rdma_h.start()
            rdma_h.wait()
            h = jnp.maximum(hbuf[...] + commh[l], 0.0)

            obuf[...] = jnp.dot(h, wouts[l][...],
                                preferred_element_type=jnp.float32)
            rdma_o = pltpu.make_async_remote_copy(
                src_ref=obuf,
                dst_ref=commo.at[l],
                send_sem=send_o.at[l],
                recv_sem=recv_o.at[l],
                device_id=x_peer,
                device_id_type=pl.DeviceIdType.MESH,
            )
            rdma_o.start()
            rdma_o.wait()
            x_cur = obuf[...] + commo[l]

        out_ref[...] = x_cur

    return pl.pallas_call(
        body,
        out_shape=jax.ShapeDtypeStruct((b, d_half), jnp.float32),
        in_specs=[pl.BlockSpec(memory_space=pltpu.VMEM)] * 7,
        out_specs=pl.BlockSpec(memory_space=pltpu.VMEM),
        scratch_shapes=[
            pltpu.VMEM((b, h_dim), jnp.float32),
            pltpu.VMEM((b, d_half), jnp.float32),
            pltpu.VMEM((N_LAYERS, b, h_dim), jnp.float32),
            pltpu.VMEM((N_LAYERS, b, d_half), jnp.float32),
            pltpu.SemaphoreType.DMA((N_LAYERS,)),
            pltpu.SemaphoreType.DMA((N_LAYERS,)),
            pltpu.SemaphoreType.DMA((N_LAYERS,)),
            pltpu.SemaphoreType.DMA((N_LAYERS,)),
        ],
        compiler_params=pltpu.CompilerParams(collective_id=0),
    )(x, Win0, Wout0, Win1, Wout1, Win2, Wout2)


# baseline (device time: 21253 ns/iter reference)
import jax
import jax.numpy as jnp
from jax import lax
from jax.experimental import pallas as pl
from jax.experimental.pallas import tpu as pltpu

N_LAYERS = 3


def kernel(x, Win0, Wout0, Win1, Wout1, Win2, Wout2):
    b, d_half = x.shape
    h_dim = Win0.shape[1]

    def body(x_ref, win0_ref, wout0_ref, win1_ref, wout1_ref, win2_ref,
             wout2_ref, out_ref,
             hbuf, obuf, commh, commo,
             send_h, recv_h, send_o, recv_o):
        my_x = lax.axis_index("x")
        my_y = lax.axis_index("y")
        y_peer = (my_x, 1 - my_y)
        x_peer = (1 - my_x, my_y)

        barrier = pltpu.get_barrier_semaphore()
        for peer in (y_peer, x_peer):
            pl.semaphore_signal(barrier, inc=1, device_id=peer,
                                device_id_type=pl.DeviceIdType.MESH)
        pl.semaphore_wait(barrier, 2)

        wins = (win0_ref, win1_ref, win2_ref)
        wouts = (wout0_ref, wout1_ref, wout2_ref)

        x_cur = x_ref[...]
        for l in range(N_LAYERS):
            hbuf[...] = jnp.dot(x_cur, wins[l][...],
                                preferred_element_type=jnp.float32)
            rdma_h = pltpu.make_async_remote_copy(
                src_ref=hbuf,
                dst_ref=commh.at[l],
                send_sem=send_h.at[l],
                recv_sem=recv_h.at[l],
                device_id=y_peer,
                device_id_type=pl.DeviceIdType.MESH,
            )
            rdma_h.start()
            rdma_h.wait()
            h = jnp.maximum(hbuf[...] + commh[l], 0.0)

            obuf[...] = jnp.dot(h, wouts[l][...],
                                preferred_element_type=jnp.float32)
            rdma_o = pltpu.make_async_remote_copy(
                src_ref=obuf,
                dst_ref=commo.at[l],
                send_sem=send_o.at[l],
                recv_sem=recv_o.at[l],
                device_id=x_peer,
                device_id_type=pl.DeviceIdType.MESH,
            )
            rdma_o.start()
            rdma_o.wait()
            x_cur = obuf[...] + commo[l]

        out_ref[...] = x_cur

    return pl.pallas_call(
        body,
        out_shape=jax.ShapeDtypeStruct((b, d_half), jnp.float32),
        in_specs=[pl.BlockSpec(memory_space=pltpu.VMEM)] * 7,
        out_specs=pl.BlockSpec(memory_space=pltpu.VMEM),
        scratch_shapes=[
            pltpu.VMEM((b, h_dim), jnp.float32),
            pltpu.VMEM((b, d_half), jnp.float32),
            pltpu.VMEM((N_LAYERS, b, h_dim), jnp.float32),
            pltpu.VMEM((N_LAYERS, b, d_half), jnp.float32),
            pltpu.SemaphoreType.DMA((N_LAYERS,)),
            pltpu.SemaphoreType.DMA((N_LAYERS,)),
            pltpu.SemaphoreType.DMA((N_LAYERS,)),
            pltpu.SemaphoreType.DMA((N_LAYERS,)),
        ],
        compiler_params=pltpu.CompilerParams(
            collective_id=0,
            vmem_limit_bytes=100 * 1024 * 1024,
        ),
    )(x, Win0, Wout0, Win1, Wout1, Win2, Wout2)
